# baseline (speedup 1.0000x reference)
import jax
import jax.numpy as jnp
from jax import lax
from jax.experimental import pallas as pl
from jax.experimental.pallas import tpu as pltpu

_BLOCK = 1024
_NCH = 8


def _prep_kernel(emb_ref, w_ih_ref, w_hh_ref, bias_ref, w_lw_ref, w_lh_ref,
                 wtop_ref, wout_ref):
    V = emb_ref.shape[0]
    H4 = w_ih_ref.shape[0]
    H = H4 // 4
    Dw = w_lw_ref.shape[0]

    col = lax.broadcasted_iota(jnp.int32, (1, H4), 1)
    is_g = (col >= 2 * H) & (col < 3 * H)
    sc = jnp.where(is_g, 1.0, 0.5).astype(jnp.float32)

    folded = lax.dot_general(emb_ref[...], w_ih_ref[...],
                             (((1,), (1,)), ((), ())),
                             preferred_element_type=jnp.float32)
    wtop_ref[0:V, :] = ((folded + bias_ref[...]) * sc).astype(jnp.bfloat16)
    wtop_ref[V:, :] = (w_hh_ref[...].T * sc).astype(jnp.bfloat16)
    wout = jnp.concatenate([w_lw_ref[...].T, w_lh_ref[...].T], axis=0)
    wout_ref[...] = jnp.pad(wout, ((0, 0), (0, 256 - Dw))).astype(jnp.bfloat16)


def _encoder_kernel(idx_ref, wemb_ref, wtop_ref, wout_ref,
                    blin_ref, out_ref):
    BLK, T = idx_ref.shape
    VH, H4 = wtop_ref.shape
    H = H4 // 4
    V = VH - H
    Dw = out_ref.shape[1]
    SB = BLK // _NCH
    ME = SB // 4

    idx = idx_ref[...]
    wtop = wtop_ref[...]
    lane_iota = lax.broadcasted_iota(jnp.int32, (SB, V), 1)

    pltpu.matmul_push_rhs(wtop[:, 0:256], staging_register=0, mxu_index=0)
    pltpu.matmul_push_rhs(wtop[:, 256:512], staging_register=0, mxu_index=1)
    wout = wout_ref[...]
    pltpu.matmul_push_rhs(wout, staging_register=1, mxu_index=0)
    pltpu.matmul_push_rhs(wout, staging_register=1, mxu_index=1)

    hs = [jnp.zeros((SB, H), jnp.bfloat16) for _ in range(_NCH)]
    cs = [jnp.zeros((SB, H), jnp.bfloat16) for _ in range(_NCH)]
    half = jnp.bfloat16(0.5)

    def _emit_acc(t, ch, lsr=None):
        rows = slice(ch * SB, (ch + 1) * SB)
        onehot = (idx[rows, t:t + 1] == lane_iota).astype(jnp.bfloat16)
        lhs = jnp.concatenate([onehot, hs[ch]], axis=1)
        pltpu.matmul_acc_lhs(ME * ch, lhs, mxu_index=0, load_staged_rhs=lsr)
        pltpu.matmul_acc_lhs(ME * ch, lhs, mxu_index=1, load_staged_rhs=lsr)

    def _pop_gates(ch):
        pre0 = pltpu.matmul_pop(ME * ch, (SB, 256), jnp.float32, mxu_index=0)
        pre1 = pltpu.matmul_pop(ME * ch, (SB, 256), jnp.float32, mxu_index=1)
        act0 = jnp.tanh(pre0.astype(jnp.bfloat16))
        act1 = jnp.tanh(pre1.astype(jnp.bfloat16))
        t_i = act0[:, 0:H]
        t_f = act0[:, H:2 * H]
        g_g = act1[:, 0:H]
        t_o = act1[:, H:2 * H]
        c = half * ((t_f * cs[ch] + cs[ch]) + (t_i * g_g + g_g))
        th = jnp.tanh(c)
        cs[ch] = c
        hs[ch] = half * (t_o * th + th)

    for ch in range(_NCH):
        _emit_acc(0, ch, lsr=0 if ch == 0 else None)
    for t in range(1, T):
        for ch in range(_NCH):
            _pop_gates(ch)
            _emit_acc(t, ch)
    for ch in range(_NCH):
        _pop_gates(ch)

    blin = blin_ref[...]
    for ch in range(_NCH):
        rows = slice(ch * SB, (ch + 1) * SB)
        comb = jnp.concatenate(
            [wemb_ref[rows, :].astype(jnp.bfloat16), hs[ch]], axis=1)
        pltpu.matmul_acc_lhs(ME * ch, comb, mxu_index=ch % 2,
                             load_staged_rhs=1 if ch < 2 else None)
    for ch in range(_NCH):
        rows = slice(ch * SB, (ch + 1) * SB)
        res = pltpu.matmul_pop(ME * ch, (SB, 256), jnp.float32,
                               mxu_index=ch % 2)
        out_ref[rows, :] = jnp.maximum(res[:, 0:Dw] + blin, 0.0)


def kernel(char_indices, word_embedding, emb_tbl, w_ih, w_hh, bias,
           w_lw, w_lh, b_lin):
    B, T = char_indices.shape
    Dw = word_embedding.shape[1]
    H4 = w_ih.shape[0]
    H = H4 // 4
    V = emb_tbl.shape[0]

    wtop, wout = pl.pallas_call(
        _prep_kernel,
        out_shape=[jax.ShapeDtypeStruct((V + H, H4), jnp.bfloat16),
                   jax.ShapeDtypeStruct((Dw + H, 256), jnp.bfloat16)],
    )(emb_tbl, w_ih, w_hh, bias, w_lw, w_lh)

    n_blk = (B + _BLOCK - 1) // _BLOCK
    Bp = n_blk * _BLOCK
    if Bp != B:
        char_indices = jnp.pad(char_indices, ((0, Bp - B), (0, 0)))
        word_embedding = jnp.pad(word_embedding, ((0, Bp - B), (0, 0)))

    out = pl.pallas_call(
        _encoder_kernel,
        out_shape=jax.ShapeDtypeStruct((Bp, Dw), jnp.float32),
        grid=(n_blk,),
        in_specs=[
            pl.BlockSpec((_BLOCK, T), lambda i: (i, 0)),
            pl.BlockSpec((_BLOCK, Dw), lambda i: (i, 0)),
            pl.BlockSpec((V + H, H4), lambda i: (0, 0)),
            pl.BlockSpec((Dw + H, 256), lambda i: (0, 0)),
            pl.BlockSpec((1, Dw), lambda i: (0, 0)),
        ],
        out_specs=pl.BlockSpec((_BLOCK, Dw), lambda i: (i, 0)),
        compiler_params=pltpu.CompilerParams(
            dimension_semantics=("parallel",)),
    )(char_indices, word_embedding, wtop, wout, b_lin)
    return out if Bp == B else out[:B]


# speedup vs baseline: 1.0619x; 1.0619x over previous
import jax
import jax.numpy as jnp
from jax import lax
from jax.experimental import pallas as pl
from jax.experimental.pallas import tpu as pltpu

_BLOCK = 2048
_NCH = 8


def _prep_kernel(emb_ref, w_ih_ref, w_hh_ref, bias_ref, w_lw_ref, w_lh_ref,
                 wtop_ref, wout_ref):
    V = emb_ref.shape[0]
    H4 = w_ih_ref.shape[0]
    H = H4 // 4

    col = lax.broadcasted_iota(jnp.int32, (1, H4), 1)
    is_g = (col >= 2 * H) & (col < 3 * H)
    sc = jnp.where(is_g, 1.0, 0.5).astype(jnp.float32)

    folded = lax.dot_general(emb_ref[...], w_ih_ref[...],
                             (((1,), (1,)), ((), ())),
                             preferred_element_type=jnp.float32)
    wtop_ref[0:V, :] = ((folded + bias_ref[...]) * sc).astype(jnp.bfloat16)
    wtop_ref[V:, :] = (w_hh_ref[...].T * sc).astype(jnp.bfloat16)
    wout_ref[...] = jnp.concatenate(
        [w_lw_ref[...].T, w_lh_ref[...].T], axis=0).astype(jnp.bfloat16)


def _encoder_kernel(idx_ref, wemb_ref, wtop_ref, wout_ref,
                    blin_ref, out_ref):
    BLK, T = idx_ref.shape
    VH, H4 = wtop_ref.shape
    H = H4 // 4
    V = VH - H
    Dw = out_ref.shape[1]
    SB = BLK // _NCH

    idx = idx_ref[...]
    wtop = wtop_ref[...]
    lane_iota = lax.broadcasted_iota(jnp.int32, (SB, V), 1)

    hs = [jnp.zeros((SB, H), jnp.bfloat16) for _ in range(_NCH)]
    cs = [jnp.zeros((SB, H), jnp.bfloat16) for _ in range(_NCH)]
    half = jnp.bfloat16(0.5)

    for t in range(T):
        for ch in range(_NCH):
            rows = slice(ch * SB, (ch + 1) * SB)
            onehot = (idx[rows, t:t + 1] == lane_iota).astype(jnp.bfloat16)
            lhs = jnp.concatenate([onehot, hs[ch]], axis=1)
            pre = jnp.dot(lhs, wtop, preferred_element_type=jnp.float32)
            act = jnp.tanh(pre.astype(jnp.bfloat16))
            t_i = act[:, 0 * H:1 * H]
            t_f = act[:, 1 * H:2 * H]
            g_g = act[:, 2 * H:3 * H]
            t_o = act[:, 3 * H:4 * H]
            c = half * ((t_f * cs[ch] + cs[ch]) + (t_i * g_g + g_g))
            th = jnp.tanh(c)
            cs[ch] = c
            hs[ch] = half * (t_o * th + th)

    wout = wout_ref[...]
    blin = blin_ref[...]
    for ch in range(_NCH):
        rows = slice(ch * SB, (ch + 1) * SB)
        comb = jnp.concatenate(
            [wemb_ref[rows, :].astype(jnp.bfloat16), hs[ch]], axis=1)
        res = jnp.dot(comb, wout, preferred_element_type=jnp.float32) + blin
        out_ref[rows, :] = jnp.maximum(res, 0.0)


def kernel(char_indices, word_embedding, emb_tbl, w_ih, w_hh, bias,
           w_lw, w_lh, b_lin):
    B, T = char_indices.shape
    Dw = word_embedding.shape[1]
    H4 = w_ih.shape[0]
    H = H4 // 4
    V = emb_tbl.shape[0]

    wtop, wout = pl.pallas_call(
        _prep_kernel,
        out_shape=[jax.ShapeDtypeStruct((V + H, H4), jnp.bfloat16),
                   jax.ShapeDtypeStruct((Dw + H, Dw), jnp.bfloat16)],
    )(emb_tbl, w_ih, w_hh, bias, w_lw, w_lh)

    n_blk = (B + _BLOCK - 1) // _BLOCK
    Bp = n_blk * _BLOCK
    if Bp != B:
        char_indices = jnp.pad(char_indices, ((0, Bp - B), (0, 0)))
        word_embedding = jnp.pad(word_embedding, ((0, Bp - B), (0, 0)))

    out = pl.pallas_call(
        _encoder_kernel,
        out_shape=jax.ShapeDtypeStruct((Bp, Dw), jnp.float32),
        grid=(n_blk,),
        in_specs=[
            pl.BlockSpec((_BLOCK, T), lambda i: (i, 0)),
            pl.BlockSpec((_BLOCK, Dw), lambda i: (i, 0)),
            pl.BlockSpec((V + H, H4), lambda i: (0, 0)),
            pl.BlockSpec((Dw + H, Dw), lambda i: (0, 0)),
            pl.BlockSpec((1, Dw), lambda i: (0, 0)),
        ],
        out_specs=pl.BlockSpec((_BLOCK, Dw), lambda i: (i, 0)),
        compiler_params=pltpu.CompilerParams(
            dimension_semantics=("parallel",)),
    )(char_indices, word_embedding, wtop, wout, b_lin)
    return out if Bp == B else out[:B]


# speedup vs baseline: 1.0682x; 1.0059x over previous
import jax
import jax.numpy as jnp
from jax import lax
from jax.experimental import pallas as pl
from jax.experimental.pallas import tpu as pltpu

_BLOCK = 2048
_NCH = 8


def _encoder_kernel(idx_ref, wemb_ref, emb_ref, w_ih_ref, w_hh_ref,
                    bias_ref, w_lw_ref, w_lh_ref, blin_ref, out_ref):
    BLK, T = idx_ref.shape
    V = emb_ref.shape[0]
    H4 = w_ih_ref.shape[0]
    H = H4 // 4
    Dw = out_ref.shape[1]
    SB = BLK // _NCH

    col = lax.broadcasted_iota(jnp.int32, (1, H4), 1)
    is_g = (col >= 2 * H) & (col < 3 * H)
    sc = jnp.where(is_g, 1.0, 0.5).astype(jnp.float32)
    folded = lax.dot_general(emb_ref[...], w_ih_ref[...],
                             (((1,), (1,)), ((), ())),
                             preferred_element_type=jnp.float32)
    wtop = jnp.concatenate(
        [(folded + bias_ref[...]) * sc, w_hh_ref[...].T * sc],
        axis=0).astype(jnp.bfloat16)
    wout = jnp.concatenate(
        [w_lw_ref[...].T, w_lh_ref[...].T], axis=0).astype(jnp.bfloat16)

    idx = idx_ref[...]
    lane_iota = lax.broadcasted_iota(jnp.int32, (SB, V), 1)

    hs = [jnp.zeros((SB, H), jnp.bfloat16) for _ in range(_NCH)]
    cs = [jnp.zeros((SB, H), jnp.bfloat16) for _ in range(_NCH)]
    half = jnp.bfloat16(0.5)

    for t in range(T):
        for ch in range(_NCH):
            rows = slice(ch * SB, (ch + 1) * SB)
            onehot = (idx[rows, t:t + 1] == lane_iota).astype(jnp.bfloat16)
            lhs = jnp.concatenate([onehot, hs[ch]], axis=1)
            pre = jnp.dot(lhs, wtop, preferred_element_type=jnp.float32)
            act = jnp.tanh(pre.astype(jnp.bfloat16))
            t_i = act[:, 0 * H:1 * H]
            t_f = act[:, 1 * H:2 * H]
            g_g = act[:, 2 * H:3 * H]
            t_o = act[:, 3 * H:4 * H]
            c = half * ((t_f * cs[ch] + cs[ch]) + (t_i * g_g + g_g))
            th = jnp.tanh(c)
            cs[ch] = c
            hs[ch] = half * (t_o * th + th)

    blin = blin_ref[...]
    for ch in range(_NCH):
        rows = slice(ch * SB, (ch + 1) * SB)
        comb = jnp.concatenate(
            [wemb_ref[rows, :].astype(jnp.bfloat16), hs[ch]], axis=1)
        res = jnp.dot(comb, wout, preferred_element_type=jnp.float32) + blin
        out_ref[rows, :] = jnp.maximum(res, 0.0)


def kernel(char_indices, word_embedding, emb_tbl, w_ih, w_hh, bias,
           w_lw, w_lh, b_lin):
    B, T = char_indices.shape
    Dw = word_embedding.shape[1]
    H4 = w_ih.shape[0]
    H = H4 // 4
    V = emb_tbl.shape[0]

    n_blk = (B + _BLOCK - 1) // _BLOCK
    Bp = n_blk * _BLOCK
    if Bp != B:
        char_indices = jnp.pad(char_indices, ((0, Bp - B), (0, 0)))
        word_embedding = jnp.pad(word_embedding, ((0, Bp - B), (0, 0)))

    out = pl.pallas_call(
        _encoder_kernel,
        out_shape=jax.ShapeDtypeStruct((Bp, Dw), jnp.float32),
        grid=(n_blk,),
        in_specs=[
            pl.BlockSpec((_BLOCK, T), lambda i: (i, 0)),
            pl.BlockSpec((_BLOCK, Dw), lambda i: (i, 0)),
            pl.BlockSpec((V, emb_tbl.shape[1]), lambda i: (0, 0)),
            pl.BlockSpec((H4, w_ih.shape[1]), lambda i: (0, 0)),
            pl.BlockSpec((H4, H), lambda i: (0, 0)),
            pl.BlockSpec((1, H4), lambda i: (0, 0)),
            pl.BlockSpec((Dw, Dw), lambda i: (0, 0)),
            pl.BlockSpec((Dw, H), lambda i: (0, 0)),
            pl.BlockSpec((1, Dw), lambda i: (0, 0)),
        ],
        out_specs=pl.BlockSpec((_BLOCK, Dw), lambda i: (i, 0)),
        compiler_params=pltpu.CompilerParams(
            dimension_semantics=("parallel",)),
    )(char_indices, word_embedding, emb_tbl, w_ih, w_hh, bias,
      w_lw, w_lh, b_lin)
    return out if Bp == B else out[:B]


# speedup vs baseline: 1.0938x; 1.0240x over previous
import jax
import jax.numpy as jnp
from jax import lax
from jax.experimental import pallas as pl
from jax.experimental.pallas import tpu as pltpu

_BLOCK = 2048
_NCH = 16


def _encoder_kernel(idx_ref, wemb_ref, emb_ref, w_ih_ref, w_hh_ref,
                    bias_ref, w_lw_ref, w_lh_ref, blin_ref, out_ref):
    BLK, T = idx_ref.shape
    V = emb_ref.shape[0]
    H4 = w_ih_ref.shape[0]
    H = H4 // 4
    Dw = out_ref.shape[1]
    SB = BLK // _NCH

    col = lax.broadcasted_iota(jnp.int32, (1, H4), 1)
    is_g = (col >= 2 * H) & (col < 3 * H)
    sc = jnp.where(is_g, 1.0, 0.5).astype(jnp.float32)
    folded = lax.dot_general(emb_ref[...], w_ih_ref[...],
                             (((1,), (1,)), ((), ())),
                             preferred_element_type=jnp.float32)
    wtop = jnp.concatenate(
        [(folded + bias_ref[...]) * sc, w_hh_ref[...].T * sc],
        axis=0).astype(jnp.bfloat16)
    wout = jnp.concatenate(
        [w_lw_ref[...].T, w_lh_ref[...].T], axis=0).astype(jnp.bfloat16)

    idx = idx_ref[...]
    lane_iota = lax.broadcasted_iota(jnp.int32, (SB, V), 1)

    hs = [jnp.zeros((SB, H), jnp.bfloat16) for _ in range(_NCH)]
    cs = [jnp.zeros((SB, H), jnp.bfloat16) for _ in range(_NCH)]
    half = jnp.bfloat16(0.5)

    for t in range(T):
        for ch in range(_NCH):
            rows = slice(ch * SB, (ch + 1) * SB)
            onehot = (idx[rows, t:t + 1] == lane_iota).astype(jnp.bfloat16)
            lhs = jnp.concatenate([onehot, hs[ch]], axis=1)
            pre = jnp.dot(lhs, wtop, preferred_element_type=jnp.float32)
            act = jnp.tanh(pre.astype(jnp.bfloat16))
            t_i = act[:, 0 * H:1 * H]
            t_f = act[:, 1 * H:2 * H]
            g_g = act[:, 2 * H:3 * H]
            t_o = act[:, 3 * H:4 * H]
            c = half * ((t_f * cs[ch] + cs[ch]) + (t_i * g_g + g_g))
            th = jnp.tanh(c)
            cs[ch] = c
            hs[ch] = half * (t_o * th + th)

    blin = blin_ref[...]
    for ch in range(_NCH):
        rows = slice(ch * SB, (ch + 1) * SB)
        comb = jnp.concatenate(
            [wemb_ref[rows, :].astype(jnp.bfloat16), hs[ch]], axis=1)
        res = jnp.dot(comb, wout, preferred_element_type=jnp.float32) + blin
        out_ref[rows, :] = jnp.maximum(res, 0.0)


def kernel(char_indices, word_embedding, emb_tbl, w_ih, w_hh, bias,
           w_lw, w_lh, b_lin):
    B, T = char_indices.shape
    Dw = word_embedding.shape[1]
    H4 = w_ih.shape[0]
    H = H4 // 4
    V = emb_tbl.shape[0]

    n_blk = (B + _BLOCK - 1) // _BLOCK
    Bp = n_blk * _BLOCK
    if Bp != B:
        char_indices = jnp.pad(char_indices, ((0, Bp - B), (0, 0)))
        word_embedding = jnp.pad(word_embedding, ((0, Bp - B), (0, 0)))

    out = pl.pallas_call(
        _encoder_kernel,
        out_shape=jax.ShapeDtypeStruct((Bp, Dw), jnp.float32),
        grid=(n_blk,),
        in_specs=[
            pl.BlockSpec((_BLOCK, T), lambda i: (i, 0)),
            pl.BlockSpec((_BLOCK, Dw), lambda i: (i, 0)),
            pl.BlockSpec((V, emb_tbl.shape[1]), lambda i: (0, 0)),
            pl.BlockSpec((H4, w_ih.shape[1]), lambda i: (0, 0)),
            pl.BlockSpec((H4, H), lambda i: (0, 0)),
            pl.BlockSpec((1, H4), lambda i: (0, 0)),
            pl.BlockSpec((Dw, Dw), lambda i: (0, 0)),
            pl.BlockSpec((Dw, H), lambda i: (0, 0)),
            pl.BlockSpec((1, Dw), lambda i: (0, 0)),
        ],
        out_specs=pl.BlockSpec((_BLOCK, Dw), lambda i: (i, 0)),
        compiler_params=pltpu.CompilerParams(
            dimension_semantics=("parallel",)),
    )(char_indices, word_embedding, emb_tbl, w_ih, w_hh, bias,
      w_lw, w_lh, b_lin)
    return out if Bp == B else out[:B]


# speedup vs baseline: 1.1038x; 1.0092x over previous
import jax
import jax.numpy as jnp
from jax import lax
from jax.experimental import pallas as pl
from jax.experimental.pallas import tpu as pltpu

_BLOCK = 4096
_NCH = 32


def _encoder_kernel(idx_ref, wemb_ref, emb_ref, w_ih_ref, w_hh_ref,
                    bias_ref, w_lw_ref, w_lh_ref, blin_ref, out_ref):
    BLK, T = idx_ref.shape
    V = emb_ref.shape[0]
    H4 = w_ih_ref.shape[0]
    H = H4 // 4
    Dw = out_ref.shape[1]
    SB = BLK // _NCH

    col = lax.broadcasted_iota(jnp.int32, (1, H4), 1)
    is_g = (col >= 2 * H) & (col < 3 * H)
    sc = jnp.where(is_g, 1.0, 0.5).astype(jnp.float32)
    folded = lax.dot_general(emb_ref[...], w_ih_ref[...],
                             (((1,), (1,)), ((), ())),
                             preferred_element_type=jnp.float32)
    wtop = jnp.concatenate(
        [(folded + bias_ref[...]) * sc, w_hh_ref[...].T * sc],
        axis=0).astype(jnp.bfloat16)
    wout = jnp.concatenate(
        [w_lw_ref[...].T, w_lh_ref[...].T], axis=0).astype(jnp.bfloat16)

    idx = idx_ref[...]
    lane_iota = lax.broadcasted_iota(jnp.int32, (SB, V), 1)

    hs = [jnp.zeros((SB, H), jnp.bfloat16) for _ in range(_NCH)]
    cs = [jnp.zeros((SB, H), jnp.bfloat16) for _ in range(_NCH)]
    half = jnp.bfloat16(0.5)

    for t in range(T):
        for ch in range(_NCH):
            rows = slice(ch * SB, (ch + 1) * SB)
            onehot = (idx[rows, t:t + 1] == lane_iota).astype(jnp.bfloat16)
            lhs = jnp.concatenate([onehot, hs[ch]], axis=1)
            pre = jnp.dot(lhs, wtop, preferred_element_type=jnp.float32)
            act = jnp.tanh(pre.astype(jnp.bfloat16))
            t_i = act[:, 0 * H:1 * H]
            t_f = act[:, 1 * H:2 * H]
            g_g = act[:, 2 * H:3 * H]
            t_o = act[:, 3 * H:4 * H]
            c = half * ((t_f * cs[ch] + cs[ch]) + (t_i * g_g + g_g))
            th = jnp.tanh(c)
            cs[ch] = c
            hs[ch] = half * (t_o * th + th)

    blin = blin_ref[...]
    for ch in range(_NCH):
        rows = slice(ch * SB, (ch + 1) * SB)
        comb = jnp.concatenate(
            [wemb_ref[rows, :].astype(jnp.bfloat16), hs[ch]], axis=1)
        res = jnp.dot(comb, wout, preferred_element_type=jnp.float32) + blin
        out_ref[rows, :] = jnp.maximum(res, 0.0)


def kernel(char_indices, word_embedding, emb_tbl, w_ih, w_hh, bias,
           w_lw, w_lh, b_lin):
    B, T = char_indices.shape
    Dw = word_embedding.shape[1]
    H4 = w_ih.shape[0]
    H = H4 // 4
    V = emb_tbl.shape[0]

    n_blk = (B + _BLOCK - 1) // _BLOCK
    Bp = n_blk * _BLOCK
    if Bp != B:
        char_indices = jnp.pad(char_indices, ((0, Bp - B), (0, 0)))
        word_embedding = jnp.pad(word_embedding, ((0, Bp - B), (0, 0)))

    out = pl.pallas_call(
        _encoder_kernel,
        out_shape=jax.ShapeDtypeStruct((Bp, Dw), jnp.float32),
        grid=(n_blk,),
        in_specs=[
            pl.BlockSpec((_BLOCK, T), lambda i: (i, 0)),
            pl.BlockSpec((_BLOCK, Dw), lambda i: (i, 0)),
            pl.BlockSpec((V, emb_tbl.shape[1]), lambda i: (0, 0)),
            pl.BlockSpec((H4, w_ih.shape[1]), lambda i: (0, 0)),
            pl.BlockSpec((H4, H), lambda i: (0, 0)),
            pl.BlockSpec((1, H4), lambda i: (0, 0)),
            pl.BlockSpec((Dw, Dw), lambda i: (0, 0)),
            pl.BlockSpec((Dw, H), lambda i: (0, 0)),
            pl.BlockSpec((1, Dw), lambda i: (0, 0)),
        ],
        out_specs=pl.BlockSpec((_BLOCK, Dw), lambda i: (i, 0)),
        compiler_params=pltpu.CompilerParams(
            dimension_semantics=("parallel",)),
    )(char_indices, word_embedding, emb_tbl, w_ih, w_hh, bias,
      w_lw, w_lh, b_lin)
    return out if Bp == B else out[:B]
